# baseline (speedup 1.0000x reference)
import functools

import jax
import jax.numpy as jnp
from jax import lax
from jax.experimental import pallas as pl
from jax.experimental import layout as jlayout
from jax.experimental.pallas import tpu as pltpu
from jax.experimental.pallas import tpu_sc as plsc

VOCAB = 1000000
SEQ = 200
DIM = 64
BATCH = 4096

NC = 2
NS = 16
NW = NC * NS

CB = 128
NBLK = VOCAB // CB
TAIL = VOCAB - NBLK * CB
BLK_PER_W = -(-NBLK // NW)

SEQ_PER_W = BATCH // NW
SEQ_PER_CHUNK = 2
NCHUNK = SEQ_PER_W // SEQ_PER_CHUNK


def _tbody(tokT_hbm, tail_hbm, tlin_hbm, in_v, out_v, sem_i, sem_o):
    wid = lax.axis_index("s") * NC + lax.axis_index("c")
    lanes = lax.iota(jnp.int32, 16)

    @pl.when(wid == 0)
    def _():
        pltpu.sync_copy(tail_hbm, out_v.at[0, pl.ds(0, TAIL * DIM)])
        pltpu.sync_copy(out_v.at[0, pl.ds(0, TAIL * DIM)],
                        tlin_hbm.at[pl.ds(NBLK * CB * DIM, TAIL * DIM)])

    def blk(k):
        return jnp.minimum(wid * BLK_PER_W + k, NBLK - 1)

    def load(k, b):
        pltpu.async_copy(tokT_hbm.at[:, pl.ds(blk(k) * CB, CB)],
                         in_v.at[b], sem_i.at[b])

    load(0, 0)

    def body(k, _):
        b = lax.rem(k, 2)
        nb = 1 - b

        @pl.when(k + 1 < BLK_PER_W)
        def _():
            load(k + 1, nb)

        pltpu.make_async_copy(tokT_hbm.at[:, pl.ds(0, CB)], in_v.at[b],
                              sem_i.at[b]).wait()

        @pl.when(k >= 2)
        def _():
            pltpu.make_async_copy(out_v.at[b], tlin_hbm.at[pl.ds(0, CB * DIM)],
                                  sem_o.at[b]).wait()

        def col(c, _):
            cvec = jnp.full((16,), c, jnp.int32)
            for d0 in range(0, DIM, 16):
                v = plsc.load_gather(in_v.at[b], [lanes + d0, cvec])
                out_v[b, pl.ds(c * DIM + d0, 16)] = v
            return ()

        lax.fori_loop(0, CB, col, ())
        pltpu.async_copy(out_v.at[b], tlin_hbm.at[pl.ds(blk(k) * CB * DIM,
                                                        CB * DIM)],
                         sem_o.at[b])
        return ()

    lax.fori_loop(0, BLK_PER_W, body, ())
    for b in range(2):
        pltpu.make_async_copy(out_v.at[b], tlin_hbm.at[pl.ds(0, CB * DIM)],
                              sem_o.at[b]).wait()


def _body(token_hbm, idx_hbm, pos_hbm, out_hbm,
          idx_v, rows_v, posrep_sh, sem_f, sem_g, sem_o):
    sid = lax.axis_index("s")
    wid = sid * NC + lax.axis_index("c")
    w_base = wid * SEQ_PER_W

    @pl.when(sid == 0)
    def _():
        for r in range(SEQ_PER_CHUNK):
            pltpu.sync_copy(pos_hbm, posrep_sh.at[r])
    plsc.subcore_barrier()

    def fill_and_gather(t, b):
        seq0 = w_base + t * SEQ_PER_CHUNK
        pltpu.sync_copy(idx_hbm.at[pl.ds(seq0, SEQ_PER_CHUNK)], idx_v.at[b])
        pltpu.async_copy(posrep_sh, rows_v.at[b], sem_f.at[b]).wait()
        for r in range(SEQ_PER_CHUNK):
            pltpu.async_copy(token_hbm.at[idx_v.at[b, r]], rows_v.at[b, r],
                             sem_g.at[b], add=True)

    fill_and_gather(0, 0)

    def chunk_body(t, _):
        b = lax.rem(t, 2)
        nb = 1 - b

        @pl.when(t + 1 < NCHUNK)
        def _():
            @pl.when(t >= 1)
            def _():
                pltpu.make_async_copy(rows_v.at[nb],
                                      out_hbm.at[pl.ds(0, SEQ_PER_CHUNK)],
                                      sem_o.at[nb]).wait()
            fill_and_gather(t + 1, nb)

        for r in range(SEQ_PER_CHUNK):
            pltpu.make_async_copy(token_hbm.at[idx_v.at[b, r]],
                                  rows_v.at[b, r], sem_g.at[b]).wait()
        seq0 = w_base + t * SEQ_PER_CHUNK
        pltpu.async_copy(rows_v.at[b],
                         out_hbm.at[pl.ds(seq0, SEQ_PER_CHUNK)],
                         sem_o.at[b])
        return ()

    lax.fori_loop(0, NCHUNK, chunk_body, ())

    for b in range(2):
        pltpu.make_async_copy(rows_v.at[b],
                              out_hbm.at[pl.ds(0, SEQ_PER_CHUNK)],
                              sem_o.at[b]).wait()


@jax.jit
def _run(idx, tokT, tail_flat, pos_table):
    mesh = plsc.VectorSubcoreMesh(core_axis_name="c", subcore_axis_name="s")

    transpose_k = functools.partial(
        pl.kernel,
        out_type=jax.ShapeDtypeStruct((VOCAB * DIM,), jnp.float32),
        mesh=mesh,
        scratch_types=[
            pltpu.VMEM((2, DIM, CB), jnp.float32),
            pltpu.VMEM((2, CB * DIM), jnp.float32),
            pltpu.SemaphoreType.DMA((2,)),
            pltpu.SemaphoreType.DMA((2,)),
        ],
        compiler_params=pltpu.CompilerParams(use_tc_tiling_on_sc=True,
                                             needs_layout_passes=False),
    )(_tbody)
    tlin = transpose_k(tokT, tail_flat)
    table = tlin.reshape(VOCAB, DIM)

    lookup_k = functools.partial(
        pl.kernel,
        out_type=jax.ShapeDtypeStruct((BATCH, SEQ, DIM), jnp.float32),
        mesh=mesh,
        scratch_types=[
            pltpu.VMEM((2, SEQ_PER_CHUNK, SEQ), jnp.int32),
            pltpu.VMEM((2, SEQ_PER_CHUNK, SEQ, DIM), jnp.float32),
            pltpu.VMEM_SHARED((SEQ_PER_CHUNK, SEQ, DIM), jnp.float32),
            pltpu.SemaphoreType.DMA((2,)),
            pltpu.SemaphoreType.DMA((2,)),
            pltpu.SemaphoreType.DMA((2,)),
        ],
        compiler_params=pltpu.CompilerParams(use_tc_tiling_on_sc=False),
    )(_body)
    out = lookup_k(table, idx, pos_table)
    return jlayout.with_layout_constraint(
        out, jlayout.Layout(major_to_minor=(0, 1, 2)))


def kernel(inputs, token_table, pos_table):
    if inputs.dtype != jnp.int32:
        inputs = inputs.astype(jnp.int32)
    tail_flat = token_table[NBLK * CB:, :].reshape(-1)
    return _run(inputs, token_table.T, tail_flat, pos_table)


# speedup vs baseline: 1.0025x; 1.0025x over previous
import functools

import jax
import jax.numpy as jnp
from jax import lax
from jax.experimental import pallas as pl
from jax.experimental import layout as jlayout
from jax.experimental.pallas import tpu as pltpu
from jax.experimental.pallas import tpu_sc as plsc

VOCAB = 1000000
SEQ = 200
DIM = 64
BATCH = 4096

NC = 2
NS = 16
NW = NC * NS

CB = 128
NBLK = VOCAB // CB
TAIL = VOCAB - NBLK * CB
BLK_PER_W = -(-NBLK // NW)

SEQ_PER_W = BATCH // NW
SEQ_PER_CHUNK = 2
NCHUNK = SEQ_PER_W // SEQ_PER_CHUNK


def _tbody(tokT_hbm, tail_hbm, tlin_hbm, in_v, out_v, sem_i, sem_o):
    wid = lax.axis_index("s") * NC + lax.axis_index("c")
    lanes = lax.iota(jnp.int32, 16)

    @pl.when(wid == 0)
    def _():
        pltpu.sync_copy(tail_hbm, out_v.at[0, pl.ds(0, TAIL * DIM)])
        pltpu.sync_copy(out_v.at[0, pl.ds(0, TAIL * DIM)],
                        tlin_hbm.at[pl.ds(NBLK * CB * DIM, TAIL * DIM)])

    def blk(k):
        return jnp.minimum(wid * BLK_PER_W + k, NBLK - 1)

    def load(k, b):
        pltpu.async_copy(tokT_hbm.at[:, pl.ds(blk(k) * CB, CB)],
                         in_v.at[b], sem_i.at[b])

    load(0, 0)

    def body(k, _):
        b = lax.rem(k, 2)
        nb = 1 - b

        @pl.when(k + 1 < BLK_PER_W)
        def _():
            load(k + 1, nb)

        pltpu.make_async_copy(tokT_hbm.at[:, pl.ds(0, CB)], in_v.at[b],
                              sem_i.at[b]).wait()

        @pl.when(k >= 2)
        def _():
            pltpu.make_async_copy(out_v.at[b], tlin_hbm.at[pl.ds(0, CB * DIM)],
                                  sem_o.at[b]).wait()

        def col(c, _):
            cvec = jnp.full((16,), c, jnp.int32)
            for d0 in range(0, DIM, 16):
                v = plsc.load_gather(in_v.at[b], [lanes + d0, cvec])
                out_v[b, pl.ds(c * DIM + d0, 16)] = v
            return ()

        lax.fori_loop(0, CB, col, (), unroll=8)
        pltpu.async_copy(out_v.at[b], tlin_hbm.at[pl.ds(blk(k) * CB * DIM,
                                                        CB * DIM)],
                         sem_o.at[b])
        return ()

    lax.fori_loop(0, BLK_PER_W, body, ())
    for b in range(2):
        pltpu.make_async_copy(out_v.at[b], tlin_hbm.at[pl.ds(0, CB * DIM)],
                              sem_o.at[b]).wait()


def _body(token_hbm, idx_hbm, pos_hbm, out_hbm,
          idx_v, rows_v, posrep_sh, sem_f, sem_g, sem_o):
    sid = lax.axis_index("s")
    wid = sid * NC + lax.axis_index("c")
    w_base = wid * SEQ_PER_W

    @pl.when(sid == 0)
    def _():
        for r in range(SEQ_PER_CHUNK):
            pltpu.sync_copy(pos_hbm, posrep_sh.at[r])
    plsc.subcore_barrier()

    def fill_and_gather(t, b):
        seq0 = w_base + t * SEQ_PER_CHUNK
        pltpu.sync_copy(idx_hbm.at[pl.ds(seq0, SEQ_PER_CHUNK)], idx_v.at[b])
        pltpu.async_copy(posrep_sh, rows_v.at[b], sem_f.at[b]).wait()
        for r in range(SEQ_PER_CHUNK):
            pltpu.async_copy(token_hbm.at[idx_v.at[b, r]], rows_v.at[b, r],
                             sem_g.at[b], add=True)

    fill_and_gather(0, 0)

    def chunk_body(t, _):
        b = lax.rem(t, 2)
        nb = 1 - b

        @pl.when(t + 1 < NCHUNK)
        def _():
            @pl.when(t >= 1)
            def _():
                pltpu.make_async_copy(rows_v.at[nb],
                                      out_hbm.at[pl.ds(0, SEQ_PER_CHUNK)],
                                      sem_o.at[nb]).wait()
            fill_and_gather(t + 1, nb)

        for r in range(SEQ_PER_CHUNK):
            pltpu.make_async_copy(token_hbm.at[idx_v.at[b, r]],
                                  rows_v.at[b, r], sem_g.at[b]).wait()
        seq0 = w_base + t * SEQ_PER_CHUNK
        pltpu.async_copy(rows_v.at[b],
                         out_hbm.at[pl.ds(seq0, SEQ_PER_CHUNK)],
                         sem_o.at[b])
        return ()

    lax.fori_loop(0, NCHUNK, chunk_body, ())

    for b in range(2):
        pltpu.make_async_copy(rows_v.at[b],
                              out_hbm.at[pl.ds(0, SEQ_PER_CHUNK)],
                              sem_o.at[b]).wait()


@jax.jit
def _run(idx, tokT, tail_flat, pos_table):
    mesh = plsc.VectorSubcoreMesh(core_axis_name="c", subcore_axis_name="s")

    transpose_k = functools.partial(
        pl.kernel,
        out_type=jax.ShapeDtypeStruct((VOCAB * DIM,), jnp.float32),
        mesh=mesh,
        scratch_types=[
            pltpu.VMEM((2, DIM, CB), jnp.float32),
            pltpu.VMEM((2, CB * DIM), jnp.float32),
            pltpu.SemaphoreType.DMA((2,)),
            pltpu.SemaphoreType.DMA((2,)),
        ],
        compiler_params=pltpu.CompilerParams(use_tc_tiling_on_sc=True,
                                             needs_layout_passes=False),
    )(_tbody)
    tlin = transpose_k(tokT, tail_flat)
    table = tlin.reshape(VOCAB, DIM)

    lookup_k = functools.partial(
        pl.kernel,
        out_type=jax.ShapeDtypeStruct((BATCH, SEQ, DIM), jnp.float32),
        mesh=mesh,
        scratch_types=[
            pltpu.VMEM((2, SEQ_PER_CHUNK, SEQ), jnp.int32),
            pltpu.VMEM((2, SEQ_PER_CHUNK, SEQ, DIM), jnp.float32),
            pltpu.VMEM_SHARED((SEQ_PER_CHUNK, SEQ, DIM), jnp.float32),
            pltpu.SemaphoreType.DMA((2,)),
            pltpu.SemaphoreType.DMA((2,)),
            pltpu.SemaphoreType.DMA((2,)),
        ],
        compiler_params=pltpu.CompilerParams(use_tc_tiling_on_sc=False),
    )(_body)
    out = lookup_k(table, idx, pos_table)
    return jlayout.with_layout_constraint(
        out, jlayout.Layout(major_to_minor=(0, 1, 2)))


def kernel(inputs, token_table, pos_table):
    if inputs.dtype != jnp.int32:
        inputs = inputs.astype(jnp.int32)
    tail_flat = token_table[NBLK * CB:, :].reshape(-1)
    return _run(inputs, token_table.T, tail_flat, pos_table)


# speedup vs baseline: 1.7397x; 1.7354x over previous
import functools

import jax
import jax.numpy as jnp
from jax import lax
from jax.experimental import pallas as pl
from jax.experimental import layout as jlayout
from jax.experimental.pallas import tpu as pltpu
from jax.experimental.pallas import tpu_sc as plsc

VOCAB = 1000000
SEQ = 200
DIM = 64
BATCH = 4096

NC = 2
NS = 16
NW = NC * NS

SEQ_PER_W = BATCH // NW
SEQ_PER_CHUNK = 2
NCHUNK = SEQ_PER_W // SEQ_PER_CHUNK


def _body(token_hbm, idx_hbm, pos_hbm, out_hbm,
          idx_v, rows_v, posrep_sh, sem_f, sem_g, sem_o):
    sid = lax.axis_index("s")
    wid = sid * NC + lax.axis_index("c")
    w_base = wid * SEQ_PER_W

    @pl.when(sid == 0)
    def _():
        for r in range(SEQ_PER_CHUNK):
            pltpu.sync_copy(pos_hbm, posrep_sh.at[r])
    plsc.subcore_barrier()

    def fill_and_gather(t, b):
        seq0 = w_base + t * SEQ_PER_CHUNK
        pltpu.sync_copy(idx_hbm.at[pl.ds(seq0, SEQ_PER_CHUNK)], idx_v.at[b])
        pltpu.async_copy(posrep_sh, rows_v.at[b], sem_f.at[b]).wait()
        for r in range(SEQ_PER_CHUNK):
            pltpu.async_copy(token_hbm.at[idx_v.at[b, r]], rows_v.at[b, r],
                             sem_g.at[b], add=True)

    fill_and_gather(0, 0)

    def chunk_body(t, _):
        b = lax.rem(t, 2)
        nb = 1 - b

        @pl.when(t + 1 < NCHUNK)
        def _():
            @pl.when(t >= 1)
            def _():
                pltpu.make_async_copy(rows_v.at[nb],
                                      out_hbm.at[pl.ds(0, SEQ_PER_CHUNK)],
                                      sem_o.at[nb]).wait()
            fill_and_gather(t + 1, nb)

        for r in range(SEQ_PER_CHUNK):
            pltpu.make_async_copy(token_hbm.at[idx_v.at[b, r]],
                                  rows_v.at[b, r], sem_g.at[b]).wait()
        seq0 = w_base + t * SEQ_PER_CHUNK
        pltpu.async_copy(rows_v.at[b],
                         out_hbm.at[pl.ds(seq0, SEQ_PER_CHUNK)],
                         sem_o.at[b])
        return ()

    lax.fori_loop(0, NCHUNK, chunk_body, ())

    for b in range(2):
        pltpu.make_async_copy(rows_v.at[b],
                              out_hbm.at[pl.ds(0, SEQ_PER_CHUNK)],
                              sem_o.at[b]).wait()


@jax.jit
def _run(idx, token_table, pos_table):
    mesh = plsc.VectorSubcoreMesh(core_axis_name="c", subcore_axis_name="s")
    f = functools.partial(
        pl.kernel,
        out_type=jax.ShapeDtypeStruct((BATCH, SEQ, DIM), jnp.float32),
        mesh=mesh,
        scratch_types=[
            pltpu.VMEM((2, SEQ_PER_CHUNK, SEQ), jnp.int32),
            pltpu.VMEM((2, SEQ_PER_CHUNK, SEQ, DIM), jnp.float32),
            pltpu.VMEM_SHARED((SEQ_PER_CHUNK, SEQ, DIM), jnp.float32),
            pltpu.SemaphoreType.DMA((2,)),
            pltpu.SemaphoreType.DMA((2,)),
            pltpu.SemaphoreType.DMA((2,)),
        ],
        compiler_params=pltpu.CompilerParams(use_tc_tiling_on_sc=False),
    )(_body)
    out = f(token_table, idx, pos_table)
    return jlayout.with_layout_constraint(
        out, jlayout.Layout(major_to_minor=(0, 1, 2)))


def kernel(inputs, token_table, pos_table):
    if inputs.dtype != jnp.int32:
        inputs = inputs.astype(jnp.int32)
    return _run(inputs, token_table, pos_table)


# speedup vs baseline: 2.0097x; 1.1552x over previous
import functools

import jax
import jax.numpy as jnp
from jax import lax
from jax.experimental import pallas as pl
from jax.experimental import layout as jlayout
from jax.experimental.pallas import tpu as pltpu
from jax.experimental.pallas import tpu_sc as plsc

VOCAB = 1000000
SEQ = 200
DIM = 64
BATCH = 4096

NC = 2
NS = 16
NW = NC * NS

SEQ_PER_W = BATCH // NW
SEQ_PER_CHUNK = 2
NCHUNK = SEQ_PER_W // SEQ_PER_CHUNK


def _body(token_hbm, idx_hbm, pos_hbm, out_hbm,
          idx_v, rows_v, posrep_sh, sem_f, sem_g, sem_o):
    sid = lax.axis_index("s")
    wid = sid * NC + lax.axis_index("c")
    w_base = wid * SEQ_PER_W

    @pl.when(sid == 0)
    def _():
        for r in range(SEQ_PER_CHUNK):
            pltpu.sync_copy(pos_hbm, posrep_sh.at[r])
    plsc.subcore_barrier()

    def fill_and_gather(t, b):
        seq0 = w_base + t * SEQ_PER_CHUNK
        pltpu.sync_copy(idx_hbm.at[pl.ds(seq0, SEQ_PER_CHUNK)], idx_v.at[b])
        pltpu.async_copy(posrep_sh, rows_v.at[b], sem_f.at[b]).wait()
        for r in range(SEQ_PER_CHUNK):
            pltpu.async_copy(token_hbm.at[idx_v.at[b, r]], rows_v.at[b, r],
                             sem_g.at[b], add=True)

    fill_and_gather(0, 0)

    def chunk_body(t, _):
        b = lax.rem(t, 2)
        nb = 1 - b

        @pl.when(t + 1 < NCHUNK)
        def _():
            @pl.when(t >= 1)
            def _():
                pltpu.make_async_copy(rows_v.at[nb],
                                      out_hbm.at[pl.ds(0, SEQ_PER_CHUNK)],
                                      sem_o.at[nb]).wait()
            fill_and_gather(t + 1, nb)

        for r in range(SEQ_PER_CHUNK):
            pltpu.make_async_copy(token_hbm.at[idx_v.at[b, r]],
                                  rows_v.at[b, r], sem_g.at[b]).wait()
        seq0 = w_base + t * SEQ_PER_CHUNK
        pltpu.async_copy(rows_v.at[b],
                         out_hbm.at[pl.ds(seq0, SEQ_PER_CHUNK)],
                         sem_o.at[b])
        return ()

    lax.fori_loop(0, NCHUNK, chunk_body, ())

    for b in range(2):
        pltpu.make_async_copy(rows_v.at[b],
                              out_hbm.at[pl.ds(0, SEQ_PER_CHUNK)],
                              sem_o.at[b]).wait()


@jax.jit
def _run(idx, token_table, pos_table):
    mesh = plsc.VectorSubcoreMesh(core_axis_name="c", subcore_axis_name="s")
    f = functools.partial(
        pl.kernel,
        out_type=jax.ShapeDtypeStruct((BATCH, SEQ, DIM), jnp.float32),
        mesh=mesh,
        scratch_types=[
            pltpu.VMEM((2, SEQ_PER_CHUNK, SEQ), jnp.int32),
            pltpu.VMEM((2, SEQ_PER_CHUNK, SEQ, DIM), jnp.float32),
            pltpu.VMEM_SHARED((SEQ_PER_CHUNK, SEQ, DIM), jnp.float32),
            pltpu.SemaphoreType.DMA((2,)),
            pltpu.SemaphoreType.DMA((2,)),
            pltpu.SemaphoreType.DMA((2,)),
        ],
        compiler_params=pltpu.CompilerParams(use_tc_tiling_on_sc=False),
    )(_body)
    return f(token_table, idx, pos_table)


def kernel(inputs, token_table, pos_table):
    if inputs.dtype != jnp.int32:
        inputs = inputs.astype(jnp.int32)
    out = _run(inputs, token_table, pos_table)
    if isinstance(out, jax.core.Tracer):
        out = jlayout.with_layout_constraint(
            out, jlayout.Layout(major_to_minor=(0, 1, 2)))
    return out
